# baseline (speedup 1.0000x reference)
import jax
import jax.numpy as jnp
from jax import lax
from jax.experimental import pallas as pl
from jax.experimental.pallas import tpu as pltpu
from jax.experimental.pallas import tpu_sc as plsc

B = 512
N = 45000
E = 100000
NC = 2
NS = 16
NW = NC * NS
E_PER_W = 3232
EP = E_PER_W * NW
C = 32
N_CH = E_PER_W // C
VPR = B // 16


def _sc_kernel(probs_t_hbm, child_hbm, parent_hbm, out_hbm,
               ci_v, pi_v, cr0_v, pr0_v, cr1_v, pr1_v, out_v, sem0, sem1):
    wid = lax.axis_index("s") * NC + lax.axis_index("c")
    ebase = wid * E_PER_W

    pltpu.sync_copy(child_hbm.at[pl.ds(ebase, E_PER_W)], ci_v)
    pltpu.sync_copy(parent_hbm.at[pl.ds(ebase, E_PER_W)], pi_v)

    def start_gather(ch, crv, prv, sem):
        pltpu.make_async_copy(
            probs_t_hbm.at[ci_v.at[pl.ds(ch * C, C)]], crv, sem).start()
        pltpu.make_async_copy(
            probs_t_hbm.at[pi_v.at[pl.ds(ch * C, C)]], prv, sem).start()

    def wait_gather(ch, crv, prv, sem):
        pltpu.make_async_copy(
            probs_t_hbm.at[ci_v.at[pl.ds(ch * C, C)]], crv, sem).wait()
        pltpu.make_async_copy(
            probs_t_hbm.at[pi_v.at[pl.ds(ch * C, C)]], prv, sem).wait()

    def chunk_compute(crv, prv, acc):
        def edge_body(e, acc):
            def vec_body(i, acc):
                c = crv[e, pl.ds(i * 16, 16)]
                p = prv[e, pl.ds(i * 16, 16)]
                return acc + jnp.maximum(c - p, jnp.zeros((16,), jnp.float32))
            return lax.fori_loop(0, VPR, vec_body, acc)
        return lax.fori_loop(0, C, edge_body, acc)

    start_gather(0, cr0_v, pr0_v, sem0)

    def pair_body(j, acc):
        start_gather(2 * j + 1, cr1_v, pr1_v, sem1)
        wait_gather(2 * j, cr0_v, pr0_v, sem0)
        acc = chunk_compute(cr0_v, pr0_v, acc)
        start_gather(2 * j + 2, cr0_v, pr0_v, sem0)
        wait_gather(2 * j + 1, cr1_v, pr1_v, sem1)
        acc = chunk_compute(cr1_v, pr1_v, acc)
        return acc

    acc = lax.fori_loop(0, (N_CH - 1) // 2, pair_body,
                        jnp.zeros((16,), jnp.float32))
    wait_gather(N_CH - 1, cr0_v, pr0_v, sem0)
    acc = chunk_compute(cr0_v, pr0_v, acc)

    out_v[...] = acc
    pltpu.sync_copy(out_v, out_hbm.at[wid])


@jax.jit
def _hierarchical_loss(probs, child, parent):
    probs_t = jnp.swapaxes(probs, 0, 1)
    pad = jnp.zeros((EP - E,), jnp.int32)
    child = jnp.concatenate([child, pad])
    parent = jnp.concatenate([parent, pad])
    mesh = plsc.VectorSubcoreMesh(core_axis_name="c", subcore_axis_name="s",
                                  num_cores=NC, num_subcores=NS)
    partials = pl.kernel(
        _sc_kernel,
        out_type=jax.ShapeDtypeStruct((NW, 16), jnp.float32),
        mesh=mesh,
        compiler_params=pltpu.CompilerParams(needs_layout_passes=False),
        scratch_types=[
            pltpu.VMEM((E_PER_W,), jnp.int32),
            pltpu.VMEM((E_PER_W,), jnp.int32),
            pltpu.VMEM((C, B), jnp.float32),
            pltpu.VMEM((C, B), jnp.float32),
            pltpu.VMEM((C, B), jnp.float32),
            pltpu.VMEM((C, B), jnp.float32),
            pltpu.VMEM((16,), jnp.float32),
            pltpu.SemaphoreType.DMA,
            pltpu.SemaphoreType.DMA,
        ],
    )(probs_t, child, parent)
    return jnp.sum(partials)


def kernel(probs, edge_index):
    child = edge_index[0].astype(jnp.int32)
    parent = edge_index[1].astype(jnp.int32)
    return _hierarchical_loss(probs, child, parent)
